# Initial kernel scaffold; baseline (speedup 1.0000x reference)
import functools

import jax
import jax.numpy as jnp
from jax import lax
from jax.experimental import pallas as pl
from jax.experimental.pallas import tpu as pltpu
from jax.experimental.pallas import tpu_sc as plsc

N = 10000
E = 320000
D = 128
NC = 2
NS = 16
NW = NC * NS
NPAD = 10240
SUBROWS = NPAD // NS
PADIDX = NPAD - 1
G = 128
EPW = 10240
EPAD = EPW * NW
GPW = EPW // G

_mesh = plsc.VectorSubcoreMesh(core_axis_name="c", subcore_axis_name="s")


@functools.partial(
    pl.kernel,
    mesh=_mesh,
    out_type=[
        jax.ShapeDtypeStruct((NC, NPAD, 16), jnp.float32),
        jax.ShapeDtypeStruct((NC, NPAD, 16), jnp.float32),
    ],
    scratch_types=[
        pltpu.VMEM((GPW, G), jnp.int32),
        pltpu.VMEM((GPW, G), jnp.int32),
        pltpu.VMEM((G, 16), jnp.float32),
        pltpu.VMEM_SHARED((NPAD, 16), jnp.float32),
        pltpu.VMEM_SHARED((NPAD, 16), jnp.float32),
    ],
)
def _deg_sc(src_hbm, dst_hbm, ones_hbm, z16_hbm, cs_hbm, cd_hbm,
            sidx, didx, ones_v, acc_s, acc_d):
    c = lax.axis_index("c")
    s = lax.axis_index("s")
    wid = s * NC + c
    pltpu.sync_copy(z16_hbm, acc_s.at[pl.ds(s * SUBROWS, SUBROWS)])
    pltpu.sync_copy(z16_hbm, acc_d.at[pl.ds(s * SUBROWS, SUBROWS)])
    pltpu.sync_copy(src_hbm.at[pl.ds(wid * GPW, GPW)], sidx)
    pltpu.sync_copy(dst_hbm.at[pl.ds(wid * GPW, GPW)], didx)
    pltpu.sync_copy(ones_hbm, ones_v)
    plsc.subcore_barrier()

    @pl.loop(0, GPW)
    def _(j):
        pltpu.sync_copy(ones_v, acc_s.at[sidx.at[j]], add=True)
        pltpu.sync_copy(ones_v, acc_d.at[didx.at[j]], add=True)

    plsc.subcore_barrier()
    pltpu.sync_copy(acc_s.at[pl.ds(s * SUBROWS, SUBROWS)],
                    cs_hbm.at[c, pl.ds(s * SUBROWS, SUBROWS)])
    pltpu.sync_copy(acc_d.at[pl.ds(s * SUBROWS, SUBROWS)],
                    cd_hbm.at[c, pl.ds(s * SUBROWS, SUBROWS)])


@functools.partial(
    pl.kernel,
    mesh=_mesh,
    out_type=jax.ShapeDtypeStruct((NC, NPAD, D), jnp.float32),
    scratch_types=[
        pltpu.VMEM((GPW, G), jnp.int32),
        pltpu.VMEM((GPW, G), jnp.int32),
        pltpu.VMEM((G, D), jnp.float32),
        pltpu.VMEM_SHARED((NPAD, D), jnp.float32),
    ],
)
def _agg_sc(tab_hbm, src_hbm, dst_hbm, z128_hbm, out_hbm,
            sidx, didx, rows, acc):
    c = lax.axis_index("c")
    s = lax.axis_index("s")
    wid = s * NC + c

    @pl.loop(0, SUBROWS // G)
    def _(i):
        pltpu.sync_copy(z128_hbm, acc.at[pl.ds(s * SUBROWS + i * G, G)])

    pltpu.sync_copy(src_hbm.at[pl.ds(wid * GPW, GPW)], sidx)
    pltpu.sync_copy(dst_hbm.at[pl.ds(wid * GPW, GPW)], didx)
    plsc.subcore_barrier()

    @pl.loop(0, GPW)
    def _(j):
        pltpu.sync_copy(tab_hbm.at[sidx.at[j]], rows)
        pltpu.sync_copy(rows, acc.at[didx.at[j]], add=True)

    plsc.subcore_barrier()

    @pl.loop(0, SUBROWS // G)
    def _(i):
        r0 = s * SUBROWS + i * G
        pltpu.sync_copy(acc.at[pl.ds(r0, G)], out_hbm.at[c, pl.ds(r0, G)])


def _dinv(cnt_blk):
    tot = cnt_blk[0] + cnt_blk[1]
    return lax.rsqrt(jnp.maximum(tot[:, 0:1], 1.0))


def _prescale_body(x_ref, cs_ref, o_ref):
    o_ref[...] = x_ref[...] * _dinv(cs_ref)


def _layer1_body(q_ref, cd_ref, cs_ref, w_ref, b_ref, o_ref):
    agg = (q_ref[0] + q_ref[1]) * _dinv(cd_ref)
    h = jnp.dot(agg, w_ref[...], preferred_element_type=jnp.float32,
                precision=lax.Precision.HIGHEST) + b_ref[...]
    o_ref[...] = jnp.maximum(h, 0.0) * _dinv(cs_ref)


def _final_body(r_ref, cd_ref, w2_ref, b2_ref, wm1_ref, bm1_ref,
                wm2_ref, bm2_ref, o_ref):
    agg = (r_ref[0] + r_ref[1]) * _dinv(cd_ref)
    h = jnp.dot(agg, w2_ref[...], preferred_element_type=jnp.float32,
                precision=lax.Precision.HIGHEST) + b2_ref[...]
    h = jnp.maximum(h, 0.0)
    h = jnp.dot(h, wm1_ref[...], preferred_element_type=jnp.float32,
                precision=lax.Precision.HIGHEST) + bm1_ref[...]
    h = jnp.maximum(h, 0.0)
    h = jnp.dot(h, wm2_ref[...], preferred_element_type=jnp.float32,
                precision=lax.Precision.HIGHEST) + bm2_ref[...]
    o_ref[...] = h


_BLK = 512
_GRID = NPAD // _BLK

_spec_rows = pl.BlockSpec((_BLK, D), lambda i: (i, 0))
_spec_part = pl.BlockSpec((NC, _BLK, D), lambda i: (0, i, 0))
_spec_cnt = pl.BlockSpec((NC, _BLK, 16), lambda i: (0, i, 0))
_spec_w = pl.BlockSpec((D, D), lambda i: (0, 0))
_spec_b = pl.BlockSpec((1, D), lambda i: (0, 0))

_prescale_tc = pl.pallas_call(
    _prescale_body,
    grid=(_GRID,),
    in_specs=[_spec_rows, _spec_cnt],
    out_specs=_spec_rows,
    out_shape=jax.ShapeDtypeStruct((NPAD, D), jnp.float32),
)

_layer1_tc = pl.pallas_call(
    _layer1_body,
    grid=(_GRID,),
    in_specs=[_spec_part, _spec_cnt, _spec_cnt, _spec_w, _spec_b],
    out_specs=_spec_rows,
    out_shape=jax.ShapeDtypeStruct((NPAD, D), jnp.float32),
)

_final_tc = pl.pallas_call(
    _final_body,
    grid=(_GRID,),
    in_specs=[_spec_part, _spec_cnt, _spec_w, _spec_b, _spec_w, _spec_b,
              _spec_w, _spec_b],
    out_specs=_spec_rows,
    out_shape=jax.ShapeDtypeStruct((NPAD, D), jnp.float32),
)


@jax.jit
def kernel(x, edge_index, W1, b1, W2, b2, Wm1, bm1, Wm2, bm2):
    src = edge_index[0].astype(jnp.int32)
    dst = edge_index[1].astype(jnp.int32)
    pad = jnp.full((EPAD - E,), PADIDX, dtype=jnp.int32)
    src2d = jnp.concatenate([src, pad]).reshape(EPAD // G, G)
    dst2d = jnp.concatenate([dst, pad]).reshape(EPAD // G, G)

    ones16 = jnp.ones((G, 16), dtype=jnp.float32)
    z16 = jnp.zeros((SUBROWS, 16), dtype=jnp.float32)
    z128 = jnp.zeros((G, D), dtype=jnp.float32)
    xpad = jnp.zeros((NPAD, D), dtype=jnp.float32).at[:N].set(x)

    cs, cd = _deg_sc(src2d, dst2d, ones16, z16)
    xs0 = _prescale_tc(xpad, cs)
    q = _agg_sc(xs0, src2d, dst2d, z128)
    xs1 = _layer1_tc(q, cd, cs, W1, b1.reshape(1, D))
    r = _agg_sc(xs1, src2d, dst2d, z128)
    out = _final_tc(r, cd, W2, b2.reshape(1, D), Wm1, bm1.reshape(1, D),
                    Wm2, bm2.reshape(1, D))
    return out[:N]


# speedup vs baseline: 2.2680x; 2.2680x over previous
import functools

import jax
import jax.numpy as jnp
from jax import lax
from jax.experimental import pallas as pl
from jax.experimental.pallas import tpu as pltpu
from jax.experimental.pallas import tpu_sc as plsc

N = 10000
E = 320000
D = 128
NC = 2
NS = 16
NW = NC * NS
NPAD = 10240
SUBROWS = NPAD // NS
PADIDX = NPAD - 1
G = 128
EPW = 10240
EPAD = EPW * NW
GPW = EPW // G

_mesh = plsc.VectorSubcoreMesh(core_axis_name="c", subcore_axis_name="s")


@functools.partial(
    pl.kernel,
    mesh=_mesh,
    out_type=jax.ShapeDtypeStruct((NC, NPAD, D), jnp.float32),
    scratch_types=[
        pltpu.VMEM((GPW, G), jnp.int32),
        pltpu.VMEM((GPW, G), jnp.int32),
        pltpu.VMEM((G, D), jnp.float32),
        pltpu.VMEM_SHARED((NPAD, D), jnp.float32),
    ],
)
def _agg_sc(tab_hbm, src_hbm, dst_hbm, z128_hbm, out_hbm,
            sidx, didx, rows, acc):
    c = lax.axis_index("c")
    s = lax.axis_index("s")
    wid = s * NC + c

    @pl.loop(0, SUBROWS // G)
    def _(i):
        pltpu.sync_copy(z128_hbm, acc.at[pl.ds(s * SUBROWS + i * G, G)])

    pltpu.sync_copy(src_hbm.at[pl.ds(wid * GPW, GPW)], sidx)
    pltpu.sync_copy(dst_hbm.at[pl.ds(wid * GPW, GPW)], didx)
    plsc.subcore_barrier()

    @pl.loop(0, GPW)
    def _(j):
        pltpu.sync_copy(tab_hbm.at[sidx.at[j]], rows)
        pltpu.sync_copy(rows, acc.at[didx.at[j]], add=True)

    plsc.subcore_barrier()

    @pl.loop(0, SUBROWS // G)
    def _(i):
        r0 = s * SUBROWS + i * G
        pltpu.sync_copy(acc.at[pl.ds(r0, G)], out_hbm.at[c, pl.ds(r0, G)])


def _dinv(cnt_blk):
    tot = cnt_blk[0] + cnt_blk[1]
    return lax.rsqrt(jnp.maximum(tot[:, 0:1], 1.0))


def _prescale_body(x_ref, cs_ref, o_ref):
    o_ref[...] = x_ref[...] * _dinv(cs_ref)


def _layer1_body(q_ref, cd_ref, cs_ref, w_ref, b_ref, o_ref):
    agg = (q_ref[0] + q_ref[1]) * _dinv(cd_ref)
    h = jnp.dot(agg, w_ref[...], preferred_element_type=jnp.float32,
                precision=lax.Precision.HIGHEST) + b_ref[...]
    o_ref[...] = jnp.maximum(h, 0.0) * _dinv(cs_ref)


def _final_body(r_ref, cd_ref, w2_ref, b2_ref, wm1_ref, bm1_ref,
                wm2_ref, bm2_ref, o_ref):
    agg = (r_ref[0] + r_ref[1]) * _dinv(cd_ref)
    h = jnp.dot(agg, w2_ref[...], preferred_element_type=jnp.float32,
                precision=lax.Precision.HIGHEST) + b2_ref[...]
    h = jnp.maximum(h, 0.0)
    h = jnp.dot(h, wm1_ref[...], preferred_element_type=jnp.float32,
                precision=lax.Precision.HIGHEST) + bm1_ref[...]
    h = jnp.maximum(h, 0.0)
    h = jnp.dot(h, wm2_ref[...], preferred_element_type=jnp.float32,
                precision=lax.Precision.HIGHEST) + bm2_ref[...]
    o_ref[...] = h


_BLK = 512
_GRID = NPAD // _BLK

_spec_rows = pl.BlockSpec((_BLK, D), lambda i: (i, 0))
_spec_part = pl.BlockSpec((NC, _BLK, D), lambda i: (0, i, 0))
_spec_w = pl.BlockSpec((D, D), lambda i: (0, 0))
_spec_b = pl.BlockSpec((1, D), lambda i: (0, 0))

_prescale_tc = pl.pallas_call(
    _prescale_body,
    grid=(_GRID,),
    in_specs=[_spec_rows, _spec_part],
    out_specs=_spec_rows,
    out_shape=jax.ShapeDtypeStruct((NPAD, D), jnp.float32),
)

_layer1_tc = pl.pallas_call(
    _layer1_body,
    grid=(_GRID,),
    in_specs=[_spec_part, _spec_part, _spec_part, _spec_w, _spec_b],
    out_specs=_spec_rows,
    out_shape=jax.ShapeDtypeStruct((NPAD, D), jnp.float32),
)

_final_tc = pl.pallas_call(
    _final_body,
    grid=(_GRID,),
    in_specs=[_spec_part, _spec_part, _spec_w, _spec_b, _spec_w, _spec_b,
              _spec_w, _spec_b],
    out_specs=_spec_rows,
    out_shape=jax.ShapeDtypeStruct((NPAD, D), jnp.float32),
)


@jax.jit
def kernel(x, edge_index, W1, b1, W2, b2, Wm1, bm1, Wm2, bm2):
    src = edge_index[0].astype(jnp.int32)
    dst = edge_index[1].astype(jnp.int32)
    pad = jnp.full((EPAD - E,), PADIDX, dtype=jnp.int32)
    src2d = jnp.concatenate([src, pad]).reshape(EPAD // G, G)
    dst2d = jnp.concatenate([dst, pad]).reshape(EPAD // G, G)

    z128 = jnp.zeros((G, D), dtype=jnp.float32)
    ones_tab = jnp.ones((NPAD, D), dtype=jnp.float32)
    xpad = jnp.zeros((NPAD, D), dtype=jnp.float32).at[:N].set(x)

    cs = _agg_sc(ones_tab, src2d, src2d, z128)
    cd = _agg_sc(ones_tab, dst2d, dst2d, z128)
    xs0 = _prescale_tc(xpad, cs)
    q = _agg_sc(xs0, src2d, dst2d, z128)
    xs1 = _layer1_tc(q, cd, cs, W1, b1.reshape(1, D))
    r = _agg_sc(xs1, src2d, dst2d, z128)
    out = _final_tc(r, cd, W2, b2.reshape(1, D), Wm1, bm1.reshape(1, D),
                    Wm2, bm2.reshape(1, D))
    return out[:N]


# speedup vs baseline: 2.4905x; 1.0981x over previous
import functools

import jax
import jax.numpy as jnp
from jax import lax
from jax.experimental import pallas as pl
from jax.experimental.pallas import tpu as pltpu
from jax.experimental.pallas import tpu_sc as plsc

N = 10000
E = 320000
D = 128
NC = 2
NS = 16
NW = NC * NS
NPAD = 10240
SUBROWS = NPAD // NS
PADIDX = NPAD - 1
G = 128
EPW = 10240
EPAD = EPW * NW
GPW = EPW // G

_mesh = plsc.VectorSubcoreMesh(core_axis_name="c", subcore_axis_name="s")


NBUF = 2
CH = 16
NCHUNK = GPW // CH


@functools.partial(
    pl.kernel,
    mesh=_mesh,
    out_type=jax.ShapeDtypeStruct((NC, NPAD, D), jnp.float32),
    scratch_types=[
        pltpu.VMEM((CH, G), jnp.int32),
        pltpu.VMEM((CH, G), jnp.int32),
    ]
    + [pltpu.VMEM((G, D), jnp.float32)] * NBUF
    + [pltpu.VMEM_SHARED((NPAD, D), jnp.float32)]
    + [pltpu.SemaphoreType.DMA] * NBUF,
)
def _agg_sc(tab_hbm, src_hbm, dst_hbm, z128_hbm, out_hbm,
            sidx, didx, r0_, r1_, acc, g0, g1):
    c = lax.axis_index("c")
    s = lax.axis_index("s")
    wid = s * NC + c
    rows = (r0_, r1_)
    gsem = (g0, g1)

    @pl.loop(0, SUBROWS // G)
    def _(i):
        pltpu.sync_copy(z128_hbm, acc.at[pl.ds(s * SUBROWS + i * G, G)])

    plsc.subcore_barrier()

    def start_g(b, j):
        pltpu.async_copy(tab_hbm.at[sidx.at[j]], rows[b], gsem[b])

    def wait_g(b, j):
        pltpu.make_async_copy(tab_hbm.at[sidx.at[j]], rows[b], gsem[b]).wait()

    @pl.loop(0, NCHUNK)
    def _(ch):
        base = wid * GPW + ch * CH
        pltpu.sync_copy(src_hbm.at[pl.ds(base, CH)], sidx)
        pltpu.sync_copy(dst_hbm.at[pl.ds(base, CH)], didx)
        start_g(0, 0)

        @pl.loop(0, CH // NBUF)
        def _(it):
            for b in range(NBUF):
                j = it * NBUF + b
                wait_g(b, j)
                if b == NBUF - 1:
                    @pl.when(it < CH // NBUF - 1)
                    def _():
                        start_g(0, j + 1)
                else:
                    start_g(b + 1, j + 1)
                pltpu.sync_copy(rows[b], acc.at[didx.at[j]], add=True)

    plsc.subcore_barrier()

    @pl.loop(0, SUBROWS // G)
    def _(i):
        r0 = s * SUBROWS + i * G
        pltpu.sync_copy(acc.at[pl.ds(r0, G)], out_hbm.at[c, pl.ds(r0, G)])


def _dinv(cnt_blk):
    tot = cnt_blk[0] + cnt_blk[1]
    return lax.rsqrt(jnp.maximum(tot[:, 0:1], 1.0))


def _prescale_body(x_ref, cs_ref, o_ref):
    o_ref[...] = x_ref[...] * _dinv(cs_ref)


def _layer1_body(q_ref, cd_ref, cs_ref, w_ref, b_ref, o_ref):
    agg = (q_ref[0] + q_ref[1]) * _dinv(cd_ref)
    h = jnp.dot(agg, w_ref[...], preferred_element_type=jnp.float32,
                precision=lax.Precision.HIGHEST) + b_ref[...]
    o_ref[...] = jnp.maximum(h, 0.0) * _dinv(cs_ref)


def _final_body(r_ref, cd_ref, w2_ref, b2_ref, wm1_ref, bm1_ref,
                wm2_ref, bm2_ref, o_ref):
    agg = (r_ref[0] + r_ref[1]) * _dinv(cd_ref)
    h = jnp.dot(agg, w2_ref[...], preferred_element_type=jnp.float32,
                precision=lax.Precision.HIGHEST) + b2_ref[...]
    h = jnp.maximum(h, 0.0)
    h = jnp.dot(h, wm1_ref[...], preferred_element_type=jnp.float32,
                precision=lax.Precision.HIGHEST) + bm1_ref[...]
    h = jnp.maximum(h, 0.0)
    h = jnp.dot(h, wm2_ref[...], preferred_element_type=jnp.float32,
                precision=lax.Precision.HIGHEST) + bm2_ref[...]
    o_ref[...] = h


_BLK = 512
_GRID = NPAD // _BLK

_spec_rows = pl.BlockSpec((_BLK, D), lambda i: (i, 0))
_spec_part = pl.BlockSpec((NC, _BLK, D), lambda i: (0, i, 0))
_spec_w = pl.BlockSpec((D, D), lambda i: (0, 0))
_spec_b = pl.BlockSpec((1, D), lambda i: (0, 0))

_prescale_tc = pl.pallas_call(
    _prescale_body,
    grid=(_GRID,),
    in_specs=[_spec_rows, _spec_part],
    out_specs=_spec_rows,
    out_shape=jax.ShapeDtypeStruct((NPAD, D), jnp.float32),
)

_layer1_tc = pl.pallas_call(
    _layer1_body,
    grid=(_GRID,),
    in_specs=[_spec_part, _spec_part, _spec_part, _spec_w, _spec_b],
    out_specs=_spec_rows,
    out_shape=jax.ShapeDtypeStruct((NPAD, D), jnp.float32),
)

_final_tc = pl.pallas_call(
    _final_body,
    grid=(_GRID,),
    in_specs=[_spec_part, _spec_part, _spec_w, _spec_b, _spec_w, _spec_b,
              _spec_w, _spec_b],
    out_specs=_spec_rows,
    out_shape=jax.ShapeDtypeStruct((NPAD, D), jnp.float32),
)


@jax.jit
def kernel(x, edge_index, W1, b1, W2, b2, Wm1, bm1, Wm2, bm2):
    src = edge_index[0].astype(jnp.int32)
    dst = edge_index[1].astype(jnp.int32)
    pad = jnp.full((EPAD - E,), PADIDX, dtype=jnp.int32)
    src2d = jnp.concatenate([src, pad]).reshape(EPAD // G, G)
    dst2d = jnp.concatenate([dst, pad]).reshape(EPAD // G, G)

    z128 = jnp.zeros((G, D), dtype=jnp.float32)
    ones_tab = jnp.ones((NPAD, D), dtype=jnp.float32)
    xpad = jnp.zeros((NPAD, D), dtype=jnp.float32).at[:N].set(x)

    cs = _agg_sc(ones_tab, src2d, src2d, z128)
    z128_dep = z128 + cs[0, :1, :1] * 0.0
    cd = _agg_sc(ones_tab, dst2d, dst2d, z128_dep)
    xs0 = _prescale_tc(xpad, cs)
    q = _agg_sc(xs0, src2d, dst2d, z128)
    xs1 = _layer1_tc(q, cd, cs, W1, b1.reshape(1, D))
    r = _agg_sc(xs1, src2d, dst2d, z128)
    out = _final_tc(r, cd, W2, b2.reshape(1, D), Wm1, bm1.reshape(1, D),
                    Wm2, bm2.reshape(1, D))
    return out[:N]


# speedup vs baseline: 3.1895x; 1.2806x over previous
import functools

import jax
import jax.numpy as jnp
from jax import lax
from jax.experimental import pallas as pl
from jax.experimental.pallas import tpu as pltpu
from jax.experimental.pallas import tpu_sc as plsc

N = 10000
E = 320000
D = 128
NC = 2
NS = 16
NW = NC * NS
NPAD = 10240
SUBROWS = NPAD // NS
PADIDX = NPAD - 1
G = 64
EPW = 10240
EPAD = EPW * NW
GPW = EPW // G

_mesh = plsc.VectorSubcoreMesh(core_axis_name="c", subcore_axis_name="s")


ZCH = 128
NBUF = 4
CH = 32
NCHUNK = GPW // CH
NITC = CH // NBUF


@functools.partial(
    pl.kernel,
    mesh=_mesh,
    out_type=jax.ShapeDtypeStruct((NC, NPAD, D), jnp.float32),
    scratch_types=[
        pltpu.VMEM((CH, G), jnp.int32),
        pltpu.VMEM((CH, G), jnp.int32),
    ]
    + [pltpu.VMEM((G, D), jnp.float32)] * NBUF
    + [pltpu.VMEM_SHARED((NPAD, D), jnp.float32)]
    + [pltpu.SemaphoreType.DMA] * (2 * NBUF),
)
def _agg_sc(tab_hbm, src_hbm, dst_hbm, z128_hbm, out_hbm,
            sidx, didx, r0_, r1_, r2_, r3_, acc,
            g0, g1, g2, g3, s0, s1, s2, s3):
    c = lax.axis_index("c")
    s = lax.axis_index("s")
    wid = s * NC + c
    rows = (r0_, r1_, r2_, r3_)
    gsem = (g0, g1, g2, g3)
    ssem = (s0, s1, s2, s3)

    @pl.loop(0, SUBROWS // ZCH)
    def _(i):
        pltpu.sync_copy(z128_hbm, acc.at[pl.ds(s * SUBROWS + i * ZCH, ZCH)])

    plsc.subcore_barrier()

    def start_g(b, j):
        pltpu.async_copy(tab_hbm.at[sidx.at[j]], rows[b], gsem[b])

    def wait_g(b, j):
        pltpu.make_async_copy(tab_hbm.at[sidx.at[j]], rows[b], gsem[b]).wait()

    def start_s(b, j):
        pltpu.async_copy(rows[b], acc.at[didx.at[j]], ssem[b], add=True)

    def wait_s(b, j):
        pltpu.make_async_copy(rows[b], acc.at[didx.at[j]], ssem[b]).wait()

    @pl.loop(0, NCHUNK)
    def _(ch):
        base = wid * GPW + ch * CH
        pltpu.sync_copy(src_hbm.at[pl.ds(base, CH)], sidx)
        pltpu.sync_copy(dst_hbm.at[pl.ds(base, CH)], didx)
        start_g(0, 0)
        start_g(1, 1)

        @pl.loop(0, NITC)
        def _(it):
            for b in range(NBUF):
                j = it * NBUF + b
                bl = (b + 2) % NBUF
                wait_g(b, j)
                start_s(b, j)
                if b < 2:
                    @pl.when(it > 0)
                    def _():
                        wait_s(bl, j)

                    start_g(bl, j + 2)
                else:
                    @pl.when(it < NITC - 1)
                    def _():
                        wait_s(bl, j)
                        start_g(bl, j + 2)

        for b in range(NBUF):
            wait_s(b, CH - NBUF + b)

    plsc.subcore_barrier()

    @pl.loop(0, SUBROWS // ZCH)
    def _(i):
        r0 = s * SUBROWS + i * ZCH
        pltpu.sync_copy(acc.at[pl.ds(r0, ZCH)], out_hbm.at[c, pl.ds(r0, ZCH)])


def _dinv(cnt_blk):
    tot = cnt_blk[0] + cnt_blk[1]
    return lax.rsqrt(jnp.maximum(tot[:, 0:1], 1.0))


def _prescale_body(x_ref, cs_ref, o_ref):
    o_ref[...] = x_ref[...] * _dinv(cs_ref)


def _layer1_body(q_ref, cd_ref, cs_ref, w_ref, b_ref, o_ref):
    agg = (q_ref[0] + q_ref[1]) * _dinv(cd_ref)
    h = jnp.dot(agg, w_ref[...], preferred_element_type=jnp.float32,
                precision=lax.Precision.HIGHEST) + b_ref[...]
    o_ref[...] = jnp.maximum(h, 0.0) * _dinv(cs_ref)


def _final_body(r_ref, cd_ref, w2_ref, b2_ref, wm1_ref, bm1_ref,
                wm2_ref, bm2_ref, o_ref):
    agg = (r_ref[0] + r_ref[1]) * _dinv(cd_ref)
    h = jnp.dot(agg, w2_ref[...], preferred_element_type=jnp.float32,
                precision=lax.Precision.HIGHEST) + b2_ref[...]
    h = jnp.maximum(h, 0.0)
    h = jnp.dot(h, wm1_ref[...], preferred_element_type=jnp.float32,
                precision=lax.Precision.HIGHEST) + bm1_ref[...]
    h = jnp.maximum(h, 0.0)
    h = jnp.dot(h, wm2_ref[...], preferred_element_type=jnp.float32,
                precision=lax.Precision.HIGHEST) + bm2_ref[...]
    o_ref[...] = h


_BLK = 512
_GRID = NPAD // _BLK

_spec_rows = pl.BlockSpec((_BLK, D), lambda i: (i, 0))
_spec_part = pl.BlockSpec((NC, _BLK, D), lambda i: (0, i, 0))
_spec_w = pl.BlockSpec((D, D), lambda i: (0, 0))
_spec_b = pl.BlockSpec((1, D), lambda i: (0, 0))

_prescale_tc = pl.pallas_call(
    _prescale_body,
    grid=(_GRID,),
    in_specs=[_spec_rows, _spec_part],
    out_specs=_spec_rows,
    out_shape=jax.ShapeDtypeStruct((NPAD, D), jnp.float32),
)

_layer1_tc = pl.pallas_call(
    _layer1_body,
    grid=(_GRID,),
    in_specs=[_spec_part, _spec_part, _spec_part, _spec_w, _spec_b],
    out_specs=_spec_rows,
    out_shape=jax.ShapeDtypeStruct((NPAD, D), jnp.float32),
)

_final_tc = pl.pallas_call(
    _final_body,
    grid=(_GRID,),
    in_specs=[_spec_part, _spec_part, _spec_w, _spec_b, _spec_w, _spec_b,
              _spec_w, _spec_b],
    out_specs=_spec_rows,
    out_shape=jax.ShapeDtypeStruct((NPAD, D), jnp.float32),
)


@jax.jit
def kernel(x, edge_index, W1, b1, W2, b2, Wm1, bm1, Wm2, bm2):
    src = edge_index[0].astype(jnp.int32)
    dst = edge_index[1].astype(jnp.int32)
    pad = jnp.full((EPAD - E,), PADIDX, dtype=jnp.int32)
    src2d = jnp.concatenate([src, pad]).reshape(EPAD // G, G)
    dst2d = jnp.concatenate([dst, pad]).reshape(EPAD // G, G)

    z128 = jnp.zeros((ZCH, D), dtype=jnp.float32)
    ones_tab = jnp.ones((NPAD, D), dtype=jnp.float32)
    xpad = jnp.zeros((NPAD, D), dtype=jnp.float32).at[:N].set(x)

    cs = _agg_sc(ones_tab, src2d, src2d, z128)
    z128_dep = z128 + cs[0, :1, :1] * 0.0
    cd = _agg_sc(ones_tab, dst2d, dst2d, z128_dep)
    xs0 = _prescale_tc(xpad, cs)
    q = _agg_sc(xs0, src2d, dst2d, z128)
    xs1 = _layer1_tc(q, cd, cs, W1, b1.reshape(1, D))
    r = _agg_sc(xs1, src2d, dst2d, z128)
    out = _final_tc(r, cd, W2, b2.reshape(1, D), Wm1, bm1.reshape(1, D),
                    Wm2, bm2.reshape(1, D))
    return out[:N]


# speedup vs baseline: 5.5520x; 1.7407x over previous
import dataclasses
import functools

import jax
import jax.numpy as jnp
from jax import lax
from jax.experimental import pallas as pl
from jax.experimental.pallas import tpu as pltpu
from jax.experimental.pallas import tpu_sc as plsc

N = 10000
E = 320000
D = 128
NC = 2
NS = 16
NW = NC * NS
NPAD = 10240
SUBROWS = NPAD // NS
PADIDX = NPAD - 1
G = 64
EPW = 10240
EPAD = EPW * NW
GPW = EPW // G

_mesh = plsc.VectorSubcoreMesh(core_axis_name="c", subcore_axis_name="s")

_cp = pltpu.CompilerParams()
if "needs_layout_passes" in pltpu.CompilerParams.__dataclass_fields__:
    _cp = dataclasses.replace(_cp, needs_layout_passes=False)


ZCH = 128
NBUF = 3
CH = 16
NCHUNK = GPW // CH


@functools.partial(
    pl.kernel,
    mesh=_mesh,
    out_type=jax.ShapeDtypeStruct((NC, NPAD, D), jnp.float32),
    scratch_types=[
        pltpu.VMEM((CH, G), jnp.int32),
        pltpu.VMEM((CH, G), jnp.int32),
    ]
    + [pltpu.VMEM((G, D), jnp.float32)] * NBUF
    + [pltpu.VMEM_SHARED((NPAD, D), jnp.float32)]
    + [pltpu.SemaphoreType.DMA] * (2 * NBUF),
)
def _agg_sc(tab_hbm, src_hbm, dst_hbm, z128_hbm, out_hbm,
            sidx, didx, r0_, r1_, r2_, acc, g0, g1, g2, s0, s1, s2):
    c = lax.axis_index("c")
    s = lax.axis_index("s")
    wid = s * NC + c
    rows = (r0_, r1_, r2_)
    gsem = (g0, g1, g2)
    ssem = (s0, s1, s2)

    @pl.loop(0, SUBROWS // ZCH)
    def _(i):
        pltpu.sync_copy(z128_hbm, acc.at[pl.ds(s * SUBROWS + i * ZCH, ZCH)])

    plsc.subcore_barrier()

    def start_g(b, j):
        pltpu.async_copy(tab_hbm.at[sidx.at[j]], rows[b], gsem[b])

    def wait_g(b, j):
        pltpu.make_async_copy(tab_hbm.at[sidx.at[j]], rows[b], gsem[b]).wait()

    def start_s(b, j):
        pltpu.async_copy(rows[b], acc.at[didx.at[j]], ssem[b], add=True)

    def wait_s(b, j):
        pltpu.make_async_copy(rows[b], acc.at[didx.at[j]], ssem[b]).wait()

    @pl.loop(0, NCHUNK)
    def _(ch):
        base = wid * GPW + ch * CH
        pltpu.sync_copy(src_hbm.at[pl.ds(base, CH)], sidx)
        pltpu.sync_copy(dst_hbm.at[pl.ds(base, CH)], didx)
        start_g(0, 0)
        start_g(1, 1)
        for j in range(CH):
            b = j % NBUF
            wait_g(b, j)
            start_s(b, j)
            if j + 2 < CH:
                bn = (j + 2) % NBUF
                if j >= 1:
                    wait_s(bn, j - 1)
                start_g(bn, j + 2)
        for j in range(CH - NBUF, CH):
            wait_s(j % NBUF, j)

    plsc.subcore_barrier()

    @pl.loop(0, SUBROWS // ZCH)
    def _(i):
        r0 = s * SUBROWS + i * ZCH
        pltpu.sync_copy(acc.at[pl.ds(r0, ZCH)], out_hbm.at[c, pl.ds(r0, ZCH)])


CCH = 16
RCH = 128
PCH = 32


@functools.partial(
    pl.kernel,
    mesh=_mesh,
    compiler_params=_cp,
    out_type=[
        jax.ShapeDtypeStruct((NC, NS, NPAD), jnp.float32),
        jax.ShapeDtypeStruct((NC, NPAD, D), jnp.float32),
    ],
    scratch_types=[
        pltpu.VMEM((CCH, G), jnp.int32),
        pltpu.VMEM((NPAD,), jnp.float32),
        pltpu.VMEM((NS, RCH), jnp.float32),
        pltpu.VMEM((PCH, D), jnp.float32),
    ],
)
def _cnt_sc(idx_hbm, bounce_hbm, out_hbm, sidx, hist, slab, pbuf):
    c = lax.axis_index("c")
    s = lax.axis_index("s")
    wid = s * NC + c
    ones = jnp.full((16,), 1.0, jnp.float32)

    @pl.loop(0, NPAD // 16)
    def _(i):
        hist[pl.ds(i * 16, 16)] = jnp.full((16,), 0.0, jnp.float32)

    @pl.loop(0, GPW // CCH)
    def _(ch):
        pltpu.sync_copy(idx_hbm.at[pl.ds(wid * GPW + ch * CCH, CCH)], sidx)

        @pl.loop(0, CCH)
        def _(j):
            for k in range(G // 16):
                plsc.addupdate_scatter(hist, [sidx[j, pl.ds(k * 16, 16)]],
                                       ones)

    pltpu.sync_copy(hist, bounce_hbm.at[c, s])
    plsc.subcore_barrier()

    iota16 = lax.iota(jnp.int32, 16)
    zeros16 = jnp.zeros((16,), jnp.int32)

    @pl.loop(0, SUBROWS // RCH)
    def _(t):
        r0 = s * SUBROWS + t * RCH
        pltpu.sync_copy(bounce_hbm.at[c, :, pl.ds(r0, RCH)], slab)
        for u in range(RCH // PCH):
            for m in range(PCH // 16):
                col = u * PCH + m * 16
                tot = slab[0, pl.ds(col, 16)]
                for w in range(1, NS):
                    tot = tot + slab[w, pl.ds(col, 16)]
                plsc.store_scatter(pbuf, [m * 16 + iota16, zeros16], tot)
            pltpu.sync_copy(pbuf, out_hbm.at[c, pl.ds(r0 + u * PCH, PCH)])


def _dinv(cnt_blk):
    tot = cnt_blk[0] + cnt_blk[1]
    return lax.rsqrt(jnp.maximum(tot[:, 0:1], 1.0))


def _prescale_body(x_ref, cs_ref, o_ref):
    o_ref[...] = x_ref[...] * _dinv(cs_ref)


def _layer1_body(q_ref, cd_ref, cs_ref, w_ref, b_ref, o_ref):
    agg = (q_ref[0] + q_ref[1]) * _dinv(cd_ref)
    h = jnp.dot(agg, w_ref[...], preferred_element_type=jnp.float32,
                precision=lax.Precision.HIGHEST) + b_ref[...]
    o_ref[...] = jnp.maximum(h, 0.0) * _dinv(cs_ref)


def _final_body(r_ref, cd_ref, w2_ref, b2_ref, wm1_ref, bm1_ref,
                wm2_ref, bm2_ref, o_ref):
    agg = (r_ref[0] + r_ref[1]) * _dinv(cd_ref)
    h = jnp.dot(agg, w2_ref[...], preferred_element_type=jnp.float32,
                precision=lax.Precision.HIGHEST) + b2_ref[...]
    h = jnp.maximum(h, 0.0)
    h = jnp.dot(h, wm1_ref[...], preferred_element_type=jnp.float32,
                precision=lax.Precision.HIGHEST) + bm1_ref[...]
    h = jnp.maximum(h, 0.0)
    h = jnp.dot(h, wm2_ref[...], preferred_element_type=jnp.float32,
                precision=lax.Precision.HIGHEST) + bm2_ref[...]
    o_ref[...] = h


_BLK = 512
_GRID = NPAD // _BLK

_spec_rows = pl.BlockSpec((_BLK, D), lambda i: (i, 0))
_spec_part = pl.BlockSpec((NC, _BLK, D), lambda i: (0, i, 0))
_spec_w = pl.BlockSpec((D, D), lambda i: (0, 0))
_spec_b = pl.BlockSpec((1, D), lambda i: (0, 0))

_prescale_tc = pl.pallas_call(
    _prescale_body,
    grid=(_GRID,),
    in_specs=[_spec_rows, _spec_part],
    out_specs=_spec_rows,
    out_shape=jax.ShapeDtypeStruct((NPAD, D), jnp.float32),
)

_layer1_tc = pl.pallas_call(
    _layer1_body,
    grid=(_GRID,),
    in_specs=[_spec_part, _spec_part, _spec_part, _spec_w, _spec_b],
    out_specs=_spec_rows,
    out_shape=jax.ShapeDtypeStruct((NPAD, D), jnp.float32),
)

_final_tc = pl.pallas_call(
    _final_body,
    grid=(_GRID,),
    in_specs=[_spec_part, _spec_part, _spec_w, _spec_b, _spec_w, _spec_b,
              _spec_w, _spec_b],
    out_specs=_spec_rows,
    out_shape=jax.ShapeDtypeStruct((NPAD, D), jnp.float32),
)


@jax.jit
def kernel(x, edge_index, W1, b1, W2, b2, Wm1, bm1, Wm2, bm2):
    src = edge_index[0].astype(jnp.int32)
    dst = edge_index[1].astype(jnp.int32)
    pad = jnp.full((EPAD - E,), PADIDX, dtype=jnp.int32)
    src2d = jnp.concatenate([src, pad]).reshape(EPAD // G, G)
    dst2d = jnp.concatenate([dst, pad]).reshape(EPAD // G, G)

    z128 = jnp.zeros((ZCH, D), dtype=jnp.float32)
    xpad = jnp.zeros((NPAD, D), dtype=jnp.float32).at[:N].set(x)

    _, cs = _cnt_sc(src2d)
    _, cd = _cnt_sc(dst2d)
    xs0 = _prescale_tc(xpad, cs)
    q = _agg_sc(xs0, src2d, dst2d, z128)
    xs1 = _layer1_tc(q, cd, cs, W1, b1.reshape(1, D))
    r = _agg_sc(xs1, src2d, dst2d, z128)
    out = _final_tc(r, cd, W2, b2.reshape(1, D), Wm1, bm1.reshape(1, D),
                    Wm2, bm2.reshape(1, D))
    return out[:N]


# speedup vs baseline: 14.2001x; 2.5577x over previous
import dataclasses
import functools

import jax
import jax.numpy as jnp
from jax import lax
from jax.experimental import pallas as pl
from jax.experimental.pallas import tpu as pltpu
from jax.experimental.pallas import tpu_sc as plsc

N = 10000
E = 320000
D = 128
NC = 2
NS = 16
NW = NC * NS
NPAD = 10240
SUBROWS = NPAD // NS
PADIDX = NPAD - 1
G = 64
EPW = 10240
EPAD = EPW * NW
GPW = EPW // G

_mesh = plsc.VectorSubcoreMesh(core_axis_name="c", subcore_axis_name="s")

_cp = pltpu.CompilerParams()
if "needs_layout_passes" in pltpu.CompilerParams.__dataclass_fields__:
    _cp = dataclasses.replace(_cp, needs_layout_passes=False)


ZCH = 128
NBUF = 3
CH = 16
NCHUNK = GPW // CH


@functools.partial(
    pl.kernel,
    mesh=_mesh,
    out_type=jax.ShapeDtypeStruct((NC, NPAD, D), jnp.float32),
    scratch_types=[
        pltpu.VMEM((CH, G), jnp.int32),
        pltpu.VMEM((CH, G), jnp.int32),
    ]
    + [pltpu.VMEM((G, D), jnp.float32)] * NBUF
    + [pltpu.VMEM_SHARED((NPAD, D), jnp.float32)]
    + [pltpu.SemaphoreType.DMA] * (2 * NBUF),
)
def _agg_sc(tab_hbm, src_hbm, dst_hbm, z128_hbm, out_hbm,
            sidx, didx, r0_, r1_, r2_, acc, g0, g1, g2, s0, s1, s2):
    c = lax.axis_index("c")
    s = lax.axis_index("s")
    wid = s * NC + c
    rows = (r0_, r1_, r2_)
    gsem = (g0, g1, g2)
    ssem = (s0, s1, s2)

    @pl.loop(0, SUBROWS // ZCH)
    def _(i):
        pltpu.sync_copy(z128_hbm, acc.at[pl.ds(s * SUBROWS + i * ZCH, ZCH)])

    plsc.subcore_barrier()

    def start_g(b, j):
        pltpu.async_copy(tab_hbm.at[sidx.at[j]], rows[b], gsem[b])

    def wait_g(b, j):
        pltpu.make_async_copy(tab_hbm.at[sidx.at[j]], rows[b], gsem[b]).wait()

    def start_s(b, j):
        pltpu.async_copy(rows[b], acc.at[didx.at[j]], ssem[b], add=True)

    def wait_s(b, j):
        pltpu.make_async_copy(rows[b], acc.at[didx.at[j]], ssem[b]).wait()

    @pl.loop(0, NCHUNK)
    def _(ch):
        base = wid * GPW + ch * CH
        pltpu.sync_copy(src_hbm.at[pl.ds(base, CH)], sidx)
        pltpu.sync_copy(dst_hbm.at[pl.ds(base, CH)], didx)
        start_g(0, 0)
        start_g(1, 1)
        for j in range(CH):
            b = j % NBUF
            wait_g(b, j)
            start_s(b, j)
            if j + 2 < CH:
                bn = (j + 2) % NBUF
                if j >= 1:
                    wait_s(bn, j - 1)
                start_g(bn, j + 2)
        for j in range(CH - NBUF, CH):
            wait_s(j % NBUF, j)

    plsc.subcore_barrier()

    @pl.loop(0, SUBROWS // ZCH)
    def _(i):
        r0 = s * SUBROWS + i * ZCH
        pltpu.sync_copy(acc.at[pl.ds(r0, ZCH)], out_hbm.at[c, pl.ds(r0, ZCH)])


CCH = 16
RCH = 128
PCH = 32


@functools.partial(
    pl.kernel,
    mesh=_mesh,
    compiler_params=_cp,
    out_type=[
        jax.ShapeDtypeStruct((NC, NS, NPAD), jnp.float32),
        jax.ShapeDtypeStruct((NC, NPAD, D), jnp.float32),
    ],
    scratch_types=[
        pltpu.VMEM((CCH, G), jnp.int32),
        pltpu.VMEM((NPAD,), jnp.float32),
        pltpu.VMEM((NS, RCH), jnp.float32),
        pltpu.VMEM((PCH, D), jnp.float32),
    ],
)
def _cnt_sc(idx_hbm, bounce_hbm, out_hbm, sidx, hist, slab, pbuf):
    c = lax.axis_index("c")
    s = lax.axis_index("s")
    wid = s * NC + c
    ones = jnp.full((16,), 1.0, jnp.float32)

    @pl.loop(0, NPAD // 16)
    def _(i):
        hist[pl.ds(i * 16, 16)] = jnp.full((16,), 0.0, jnp.float32)

    @pl.loop(0, GPW // CCH)
    def _(ch):
        pltpu.sync_copy(idx_hbm.at[pl.ds(wid * GPW + ch * CCH, CCH)], sidx)

        @pl.loop(0, CCH)
        def _(j):
            for k in range(G // 16):
                plsc.addupdate_scatter(hist, [sidx[j, pl.ds(k * 16, 16)]],
                                       ones)

    pltpu.sync_copy(hist, bounce_hbm.at[c, s])
    plsc.subcore_barrier()

    iota16 = lax.iota(jnp.int32, 16)
    zeros16 = jnp.zeros((16,), jnp.int32)

    @pl.loop(0, SUBROWS // RCH)
    def _(t):
        r0 = s * SUBROWS + t * RCH
        pltpu.sync_copy(bounce_hbm.at[c, :, pl.ds(r0, RCH)], slab)
        for u in range(RCH // PCH):
            for m in range(PCH // 16):
                col = u * PCH + m * 16
                tot = slab[0, pl.ds(col, 16)]
                for w in range(1, NS):
                    tot = tot + slab[w, pl.ds(col, 16)]
                plsc.store_scatter(pbuf, [m * 16 + iota16, zeros16], tot)
            pltpu.sync_copy(pbuf, out_hbm.at[c, pl.ds(r0 + u * PCH, PCH)])


def _dinv(cnt_blk):
    tot = cnt_blk[0] + cnt_blk[1]
    return lax.rsqrt(jnp.maximum(tot[:, 0:1], 1.0))


def _prescale_body(x_ref, cs_ref, o_ref):
    o_ref[...] = x_ref[...] * _dinv(cs_ref)


def _layer1_body(q_ref, cd_ref, cs_ref, w_ref, b_ref, o_ref):
    agg = (q_ref[0] + q_ref[1]) * _dinv(cd_ref)
    h = jnp.dot(agg, w_ref[...], preferred_element_type=jnp.float32,
                precision=lax.Precision.HIGHEST) + b_ref[...]
    o_ref[...] = jnp.maximum(h, 0.0) * _dinv(cs_ref)


def _final_body(r_ref, cd_ref, w2_ref, b2_ref, wm1_ref, bm1_ref,
                wm2_ref, bm2_ref, o_ref):
    agg = (r_ref[0] + r_ref[1]) * _dinv(cd_ref)
    h = jnp.dot(agg, w2_ref[...], preferred_element_type=jnp.float32,
                precision=lax.Precision.HIGHEST) + b2_ref[...]
    h = jnp.maximum(h, 0.0)
    h = jnp.dot(h, wm1_ref[...], preferred_element_type=jnp.float32,
                precision=lax.Precision.HIGHEST) + bm1_ref[...]
    h = jnp.maximum(h, 0.0)
    h = jnp.dot(h, wm2_ref[...], preferred_element_type=jnp.float32,
                precision=lax.Precision.HIGHEST) + bm2_ref[...]
    o_ref[...] = h


_BLK = 512
_GRID = NPAD // _BLK

_spec_rows = pl.BlockSpec((_BLK, D), lambda i: (i, 0))
_spec_part = pl.BlockSpec((NC, _BLK, D), lambda i: (0, i, 0))
_spec_w = pl.BlockSpec((D, D), lambda i: (0, 0))
_spec_b = pl.BlockSpec((1, D), lambda i: (0, 0))

_prescale_tc = pl.pallas_call(
    _prescale_body,
    grid=(_GRID,),
    in_specs=[_spec_rows, _spec_part],
    out_specs=_spec_rows,
    out_shape=jax.ShapeDtypeStruct((NPAD, D), jnp.float32),
)

_layer1_tc = pl.pallas_call(
    _layer1_body,
    grid=(_GRID,),
    in_specs=[_spec_part, _spec_part, _spec_part, _spec_w, _spec_b],
    out_specs=_spec_rows,
    out_shape=jax.ShapeDtypeStruct((NPAD, D), jnp.float32),
)

_final_tc = pl.pallas_call(
    _final_body,
    grid=(_GRID,),
    in_specs=[_spec_part, _spec_part, _spec_w, _spec_b, _spec_w, _spec_b,
              _spec_w, _spec_b],
    out_specs=_spec_rows,
    out_shape=jax.ShapeDtypeStruct((NPAD, D), jnp.float32),
)


@jax.jit
def kernel(x, edge_index, W1, b1, W2, b2, Wm1, bm1, Wm2, bm2):
    src = edge_index[0].astype(jnp.int32)
    dst = edge_index[1].astype(jnp.int32)
    pad = N + jnp.arange(EPAD - E, dtype=jnp.int32) % (NPAD - N)
    src2d = jnp.concatenate([src, pad]).reshape(EPAD // G, G)
    dst2d = jnp.concatenate([dst, pad]).reshape(EPAD // G, G)

    z128 = jnp.zeros((ZCH, D), dtype=jnp.float32)
    xpad = jnp.zeros((NPAD, D), dtype=jnp.float32).at[:N].set(x)

    _, cs = _cnt_sc(src2d)
    _, cd = _cnt_sc(dst2d)
    xs0 = _prescale_tc(xpad, cs)
    q = _agg_sc(xs0, src2d, dst2d, z128)
    xs1 = _layer1_tc(q, cd, cs, W1, b1.reshape(1, D))
    r = _agg_sc(xs1, src2d, dst2d, z128)
    out = _final_tc(r, cd, W2, b2.reshape(1, D), Wm1, bm1.reshape(1, D),
                    Wm2, bm2.reshape(1, D))
    return out[:N]


# speedup vs baseline: 14.8248x; 1.0440x over previous
import dataclasses
import functools

import jax
import jax.numpy as jnp
from jax import lax
from jax.experimental import pallas as pl
from jax.experimental.pallas import tpu as pltpu
from jax.experimental.pallas import tpu_sc as plsc

N = 10000
E = 320000
D = 128
NC = 2
NS = 16
NW = NC * NS
NPAD = 10240
SUBROWS = NPAD // NS
PADIDX = NPAD - 1
G = 64
EPW = 10240
EPAD = EPW * NW
GPW = EPW // G

_mesh = plsc.VectorSubcoreMesh(core_axis_name="c", subcore_axis_name="s")

_cp = pltpu.CompilerParams()
if "needs_layout_passes" in pltpu.CompilerParams.__dataclass_fields__:
    _cp = dataclasses.replace(_cp, needs_layout_passes=False)


ZCH = 128
NBUF = 3
CH = 32
NCHUNK = GPW // CH


@functools.partial(
    pl.kernel,
    mesh=_mesh,
    out_type=jax.ShapeDtypeStruct((NC, NPAD, D), jnp.float32),
    scratch_types=[
        pltpu.VMEM((CH, G), jnp.int32),
        pltpu.VMEM((CH, G), jnp.int32),
    ]
    + [pltpu.VMEM((G, D), jnp.float32)] * NBUF
    + [pltpu.VMEM_SHARED((NPAD, D), jnp.float32)]
    + [pltpu.SemaphoreType.DMA] * (2 * NBUF),
)
def _agg_sc(tab_hbm, src_hbm, dst_hbm, z128_hbm, out_hbm,
            sidx, didx, r0_, r1_, r2_, acc, g0, g1, g2, s0, s1, s2):
    c = lax.axis_index("c")
    s = lax.axis_index("s")
    wid = s * NC + c
    rows = (r0_, r1_, r2_)
    gsem = (g0, g1, g2)
    ssem = (s0, s1, s2)

    @pl.loop(0, SUBROWS // ZCH)
    def _(i):
        pltpu.sync_copy(z128_hbm, acc.at[pl.ds(s * SUBROWS + i * ZCH, ZCH)])

    plsc.subcore_barrier()

    def start_g(b, j):
        pltpu.async_copy(tab_hbm.at[sidx.at[j]], rows[b], gsem[b])

    def wait_g(b, j):
        pltpu.make_async_copy(tab_hbm.at[sidx.at[j]], rows[b], gsem[b]).wait()

    def start_s(b, j):
        pltpu.async_copy(rows[b], acc.at[didx.at[j]], ssem[b], add=True)

    def wait_s(b, j):
        pltpu.make_async_copy(rows[b], acc.at[didx.at[j]], ssem[b]).wait()

    @pl.loop(0, NCHUNK)
    def _(ch):
        base = wid * GPW + ch * CH
        pltpu.sync_copy(src_hbm.at[pl.ds(base, CH)], sidx)
        pltpu.sync_copy(dst_hbm.at[pl.ds(base, CH)], didx)
        start_g(0, 0)
        start_g(1, 1)
        for j in range(CH):
            b = j % NBUF
            wait_g(b, j)
            start_s(b, j)
            if j + 2 < CH:
                bn = (j + 2) % NBUF
                if j >= 1:
                    wait_s(bn, j - 1)
                start_g(bn, j + 2)
        for j in range(CH - NBUF, CH):
            wait_s(j % NBUF, j)

    plsc.subcore_barrier()

    @pl.loop(0, SUBROWS // ZCH)
    def _(i):
        r0 = s * SUBROWS + i * ZCH
        pltpu.sync_copy(acc.at[pl.ds(r0, ZCH)], out_hbm.at[c, pl.ds(r0, ZCH)])


CCH = 16
RCH = 128
PCH = 32


@functools.partial(
    pl.kernel,
    mesh=_mesh,
    compiler_params=_cp,
    out_type=[
        jax.ShapeDtypeStruct((NC, NS, NPAD), jnp.float32),
        jax.ShapeDtypeStruct((NC, NPAD, D), jnp.float32),
    ],
    scratch_types=[
        pltpu.VMEM((CCH, G), jnp.int32),
        pltpu.VMEM((NPAD,), jnp.float32),
        pltpu.VMEM((NS, RCH), jnp.float32),
        pltpu.VMEM((PCH, D), jnp.float32),
    ],
)
def _cnt_sc(idx_hbm, bounce_hbm, out_hbm, sidx, hist, slab, pbuf):
    c = lax.axis_index("c")
    s = lax.axis_index("s")
    wid = s * NC + c
    ones = jnp.full((16,), 1.0, jnp.float32)

    @pl.loop(0, NPAD // 16)
    def _(i):
        hist[pl.ds(i * 16, 16)] = jnp.full((16,), 0.0, jnp.float32)

    @pl.loop(0, GPW // CCH)
    def _(ch):
        pltpu.sync_copy(idx_hbm.at[pl.ds(wid * GPW + ch * CCH, CCH)], sidx)

        @pl.loop(0, CCH)
        def _(j):
            for k in range(G // 16):
                plsc.addupdate_scatter(hist, [sidx[j, pl.ds(k * 16, 16)]],
                                       ones)

    pltpu.sync_copy(hist, bounce_hbm.at[c, s])
    plsc.subcore_barrier()

    iota16 = lax.iota(jnp.int32, 16)
    zeros16 = jnp.zeros((16,), jnp.int32)

    @pl.loop(0, SUBROWS // RCH)
    def _(t):
        r0 = s * SUBROWS + t * RCH
        pltpu.sync_copy(bounce_hbm.at[c, :, pl.ds(r0, RCH)], slab)
        for u in range(RCH // PCH):
            for m in range(PCH // 16):
                col = u * PCH + m * 16
                tot = slab[0, pl.ds(col, 16)]
                for w in range(1, NS):
                    tot = tot + slab[w, pl.ds(col, 16)]
                plsc.store_scatter(pbuf, [m * 16 + iota16, zeros16], tot)
            pltpu.sync_copy(pbuf, out_hbm.at[c, pl.ds(r0 + u * PCH, PCH)])


def _dinv(cnt_blk):
    tot = cnt_blk[0] + cnt_blk[1]
    return lax.rsqrt(jnp.maximum(tot[:, 0:1], 1.0))


def _prescale_body(x_ref, cs_ref, o_ref):
    o_ref[...] = x_ref[...] * _dinv(cs_ref)


def _layer1_body(q_ref, cd_ref, cs_ref, w_ref, b_ref, o_ref):
    agg = (q_ref[0] + q_ref[1]) * _dinv(cd_ref)
    h = jnp.dot(agg, w_ref[...], preferred_element_type=jnp.float32,
                precision=lax.Precision.HIGHEST) + b_ref[...]
    o_ref[...] = jnp.maximum(h, 0.0) * _dinv(cs_ref)


def _final_body(r_ref, cd_ref, w2_ref, b2_ref, wm1_ref, bm1_ref,
                wm2_ref, bm2_ref, o_ref):
    agg = (r_ref[0] + r_ref[1]) * _dinv(cd_ref)
    h = jnp.dot(agg, w2_ref[...], preferred_element_type=jnp.float32,
                precision=lax.Precision.HIGHEST) + b2_ref[...]
    h = jnp.maximum(h, 0.0)
    h = jnp.dot(h, wm1_ref[...], preferred_element_type=jnp.float32,
                precision=lax.Precision.HIGHEST) + bm1_ref[...]
    h = jnp.maximum(h, 0.0)
    h = jnp.dot(h, wm2_ref[...], preferred_element_type=jnp.float32,
                precision=lax.Precision.HIGHEST) + bm2_ref[...]
    o_ref[...] = h


_BLK = 512
_GRID = NPAD // _BLK

_spec_rows = pl.BlockSpec((_BLK, D), lambda i: (i, 0))
_spec_part = pl.BlockSpec((NC, _BLK, D), lambda i: (0, i, 0))
_spec_w = pl.BlockSpec((D, D), lambda i: (0, 0))
_spec_b = pl.BlockSpec((1, D), lambda i: (0, 0))

_prescale_tc = pl.pallas_call(
    _prescale_body,
    grid=(_GRID,),
    in_specs=[_spec_rows, _spec_part],
    out_specs=_spec_rows,
    out_shape=jax.ShapeDtypeStruct((NPAD, D), jnp.float32),
)

_layer1_tc = pl.pallas_call(
    _layer1_body,
    grid=(_GRID,),
    in_specs=[_spec_part, _spec_part, _spec_part, _spec_w, _spec_b],
    out_specs=_spec_rows,
    out_shape=jax.ShapeDtypeStruct((NPAD, D), jnp.float32),
)

_final_tc = pl.pallas_call(
    _final_body,
    grid=(_GRID,),
    in_specs=[_spec_part, _spec_part, _spec_w, _spec_b, _spec_w, _spec_b,
              _spec_w, _spec_b],
    out_specs=_spec_rows,
    out_shape=jax.ShapeDtypeStruct((NPAD, D), jnp.float32),
)


@jax.jit
def kernel(x, edge_index, W1, b1, W2, b2, Wm1, bm1, Wm2, bm2):
    src = edge_index[0].astype(jnp.int32)
    dst = edge_index[1].astype(jnp.int32)
    pad = N + jnp.arange(EPAD - E, dtype=jnp.int32) % (NPAD - N)
    src2d = jnp.concatenate([src, pad]).reshape(EPAD // G, G)
    dst2d = jnp.concatenate([dst, pad]).reshape(EPAD // G, G)

    z128 = jnp.zeros((ZCH, D), dtype=jnp.float32)
    xpad = jnp.zeros((NPAD, D), dtype=jnp.float32).at[:N].set(x)

    _, cs = _cnt_sc(src2d)
    _, cd = _cnt_sc(dst2d)
    xs0 = _prescale_tc(xpad, cs)
    q = _agg_sc(xs0, src2d, dst2d, z128)
    xs1 = _layer1_tc(q, cd, cs, W1, b1.reshape(1, D))
    r = _agg_sc(xs1, src2d, dst2d, z128)
    out = _final_tc(r, cd, W2, b2.reshape(1, D), Wm1, bm1.reshape(1, D),
                    Wm2, bm2.reshape(1, D))
    return out[:N]
